# Initial kernel scaffold; baseline (speedup 1.0000x reference)
import jax
import jax.numpy as jnp
from jax import lax
from jax.experimental import pallas as pl
from jax.experimental.pallas import tpu as pltpu
from jax.experimental.pallas import tpu_sc as plsc

_VOCAB = 1000000
_EMBED = 64
_OUT = 128
_BATCH = 4096
_SEQ = 200

_NC = 2
_NS = 16
_NW = _NC * _NS
_BPW = _BATCH // _NW
_IDXW = _BPW * _SEQ
_CH0 = 104
_CH1 = _SEQ - _CH0


def _pool_body(x_hbm, table_hbm, out_hbm, idx_v, rows0, rows1, out_v,
               sem0, sem1):
    wid = lax.axis_index("s") * _NC + lax.axis_index("c")
    idx_base = wid * _IDXW

    pltpu.sync_copy(x_hbm.at[pl.ds(idx_base, _IDXW)], idx_v)

    def fire(e, rows_ref, sem):
        off = e * _SEQ
        pltpu.async_copy(
            table_hbm.at[idx_v.at[pl.ds(off, _CH0)]],
            rows_ref.at[pl.ds(0, _CH0)], sem)
        pltpu.async_copy(
            table_hbm.at[idx_v.at[pl.ds(off + _CH0, _CH1)]],
            rows_ref.at[pl.ds(_CH0, _CH1)], sem)

    def wait(rows_ref, sem):
        pltpu.make_async_copy(
            table_hbm.at[pl.ds(0, _SEQ)], rows_ref, sem).wait()

    def accum(rows_ref, e):
        def body(s, carry):
            a0, a1, a2, a3 = carry
            return (a0 + rows_ref[s, pl.ds(0, 16)],
                    a1 + rows_ref[s, pl.ds(16, 16)],
                    a2 + rows_ref[s, pl.ds(32, 16)],
                    a3 + rows_ref[s, pl.ds(48, 16)])
        z = jnp.zeros((16,), jnp.float32)
        a0, a1, a2, a3 = lax.fori_loop(0, _SEQ, body, (z, z, z, z),
                                       unroll=4)
        out_v[e, pl.ds(0, 16)] = a0
        out_v[e, pl.ds(16, 16)] = a1
        out_v[e, pl.ds(32, 16)] = a2
        out_v[e, pl.ds(48, 16)] = a3

    fire(0, rows0, sem0)
    fire(1, rows1, sem1)

    def step(k, _):
        e0 = 2 * k
        wait(rows0, sem0)
        accum(rows0, e0)

        @pl.when(k < _BPW // 2 - 1)
        def _():
            fire(e0 + 2, rows0, sem0)

        wait(rows1, sem1)
        accum(rows1, e0 + 1)

        @pl.when(k < _BPW // 2 - 1)
        def _():
            fire(e0 + 3, rows1, sem1)

        return 0

    lax.fori_loop(0, _BPW // 2, step, 0)

    pltpu.sync_copy(out_v, out_hbm.at[pl.ds(wid * _BPW, _BPW)])


_pool = pl.kernel(
    _pool_body,
    out_type=jax.ShapeDtypeStruct((_BATCH, _EMBED), jnp.float32),
    mesh=plsc.VectorSubcoreMesh(core_axis_name="c", subcore_axis_name="s",
                                num_cores=_NC, num_subcores=_NS),
    scratch_types=[
        pltpu.VMEM((_IDXW,), jnp.int32),
        pltpu.VMEM((_SEQ, _EMBED), jnp.float32),
        pltpu.VMEM((_SEQ, _EMBED), jnp.float32),
        pltpu.VMEM((_BPW, _EMBED), jnp.float32),
        pltpu.SemaphoreType.DMA,
        pltpu.SemaphoreType.DMA,
    ],
)


def _mlp_body(p_ref, w_ref, b_ref, o_ref):
    pooled = p_ref[...] * (1.0 / _SEQ)
    o_ref[...] = (
        jnp.dot(pooled, w_ref[...], preferred_element_type=jnp.float32)
        + b_ref[...])


_mlp = pl.pallas_call(
    _mlp_body,
    out_shape=jax.ShapeDtypeStruct((_BATCH, _OUT), jnp.float32),
)


@jax.jit
def kernel(x, table, W1, b1):
    x_flat = x.reshape(-1).astype(jnp.int32)
    sums = _pool(x_flat, table)
    return _mlp(sums, W1, b1.reshape(1, _OUT))


# speedup vs baseline: 1.0526x; 1.0526x over previous
import jax
import jax.numpy as jnp
from jax import lax
from jax.experimental import pallas as pl
from jax.experimental.pallas import tpu as pltpu
from jax.experimental.pallas import tpu_sc as plsc

_VOCAB = 1000000
_EMBED = 64
_OUT = 128
_BATCH = 4096
_SEQ = 200

_NC = 2
_NS = 16
_NW = _NC * _NS
_BPW = _BATCH // _NW
_IDXW = _BPW * _SEQ
_CH0 = 104
_CH1 = _SEQ - _CH0


def _pool_body(x_hbm, table_hbm, out_hbm, idx_v, rows0, rows1, out_v,
               sem0, sem1):
    wid = lax.axis_index("s") * _NC + lax.axis_index("c")
    idx_base = wid * _IDXW

    pltpu.sync_copy(x_hbm.at[pl.ds(idx_base, _IDXW)], idx_v)

    def fire(e, rows_ref, sem):
        off = e * _SEQ
        pltpu.async_copy(
            table_hbm.at[idx_v.at[pl.ds(off, _CH0)]],
            rows_ref.at[pl.ds(0, _CH0)], sem)
        pltpu.async_copy(
            table_hbm.at[idx_v.at[pl.ds(off + _CH0, _CH1)]],
            rows_ref.at[pl.ds(_CH0, _CH1)], sem)

    def wait(rows_ref, sem):
        pltpu.make_async_copy(
            table_hbm.at[pl.ds(0, _SEQ)], rows_ref, sem).wait()

    def accum(rows_ref, e):
        def body(s, carry):
            a0, a1, a2, a3 = carry
            return (a0 + rows_ref[s, pl.ds(0, 16)],
                    a1 + rows_ref[s, pl.ds(16, 16)],
                    a2 + rows_ref[s, pl.ds(32, 16)],
                    a3 + rows_ref[s, pl.ds(48, 16)])
        z = jnp.zeros((16,), jnp.float32)
        a0, a1, a2, a3 = lax.fori_loop(0, _SEQ, body, (z, z, z, z),
                                       unroll=4)
        out_v[e, pl.ds(0, 16)] = a0
        out_v[e, pl.ds(16, 16)] = a1
        out_v[e, pl.ds(32, 16)] = a2
        out_v[e, pl.ds(48, 16)] = a3

    fire(0, rows0, sem0)
    fire(1, rows1, sem1)

    def step(k, _):
        e0 = 2 * k
        wait(rows0, sem0)
        accum(rows0, e0)

        @pl.when(k < _BPW // 2 - 1)
        def _():
            fire(e0 + 2, rows0, sem0)

        wait(rows1, sem1)
        accum(rows1, e0 + 1)

        @pl.when(k < _BPW // 2 - 1)
        def _():
            fire(e0 + 3, rows1, sem1)

        return 0

    lax.fori_loop(0, _BPW // 2, step, 0)

    pltpu.sync_copy(out_v, out_hbm.at[pl.ds(wid * _BPW, _BPW)])


_pool = pl.kernel(
    _pool_body,
    out_type=jax.ShapeDtypeStruct((_BATCH, _EMBED), jnp.float32),
    mesh=plsc.VectorSubcoreMesh(core_axis_name="c", subcore_axis_name="s",
                                num_cores=_NC, num_subcores=_NS),
    compiler_params=pltpu.CompilerParams(use_tc_tiling_on_sc=False),
    scratch_types=[
        pltpu.VMEM((_IDXW,), jnp.int32),
        pltpu.VMEM((_SEQ, _EMBED), jnp.float32),
        pltpu.VMEM((_SEQ, _EMBED), jnp.float32),
        pltpu.VMEM((_BPW, _EMBED), jnp.float32),
        pltpu.SemaphoreType.DMA,
        pltpu.SemaphoreType.DMA,
    ],
)


def _mlp_body(p_ref, w_ref, b_ref, o_ref):
    pooled = p_ref[...] * (1.0 / _SEQ)
    o_ref[...] = (
        jnp.dot(pooled, w_ref[...], preferred_element_type=jnp.float32)
        + b_ref[...])


_mlp = pl.pallas_call(
    _mlp_body,
    out_shape=jax.ShapeDtypeStruct((_BATCH, _OUT), jnp.float32),
)


@jax.jit
def kernel(x, table, W1, b1):
    x_flat = x.reshape(-1).astype(jnp.int32)
    sums = _pool(x_flat, table)
    return _mlp(sums, W1, b1.reshape(1, _OUT))
